# baseline (speedup 1.0000x reference)
import jax
import jax.numpy as jnp
from jax.experimental import pallas as pl


def _fused_kernel(wq_ref, cbt_ref, idx_ref, wet_ref, oh_ref):
    NCc = wq_ref.shape[0]
    for i in range(NCc):
        wq = wq_ref[i]
        cbt = cbt_ref[i]
        B, D = wq.shape
        K = cbt.shape[1]
        wq_sq = jnp.sum(wq * wq, axis=1)
        cb_sq = jnp.sum(cbt * cbt, axis=0)
        cross = jax.lax.dot_general(
            wq, cbt, (((1,), (0,)), ((), ())),
            preferred_element_type=jnp.float32,
            precision=jax.lax.Precision.DEFAULT)
        d = (wq_sq[:, None] + cb_sq[None, :]) - 2.0 * cross
        idx = jnp.argmin(d, axis=1).astype(jnp.int32)
        idx_ref[i, 0] = idx
        kio = jax.lax.broadcasted_iota(jnp.int32, (B, K), 1)
        oh_ref[:, i, :] = (kio == idx[:, None]).astype(oh_ref.dtype)
        kio_t = jax.lax.broadcasted_iota(jnp.int32, (K, 1), 0)
        oh_t = (kio_t == idx[None, :]).astype(jnp.float32)
        wet_ref[i] = jax.lax.dot_general(
            cbt, oh_t, (((1,), (0,)), ((), ())),
            preferred_element_type=jnp.float32,
            precision=jax.lax.Precision.DEFAULT)


def kernel(w_q, codebook):
    B, C, D = w_q.shape
    K = codebook.shape[1]
    wq_t = jnp.transpose(w_q, (1, 0, 2))
    cb_t = jnp.transpose(codebook, (0, 2, 1))
    NCc = 8
    idx_t, we_t, one_hot = pl.pallas_call(
        _fused_kernel,
        grid=(C // NCc,),
        in_specs=[
            pl.BlockSpec((NCc, B, D), lambda c: (c, 0, 0)),
            pl.BlockSpec((NCc, D, K), lambda c: (c, 0, 0)),
        ],
        out_specs=[
            pl.BlockSpec((NCc, 1, B), lambda c: (c, 0, 0)),
            pl.BlockSpec((NCc, D, B), lambda c: (c, 0, 0)),
            pl.BlockSpec((B, NCc, K), lambda c: (0, c, 0)),
        ],
        out_shape=[
            jax.ShapeDtypeStruct((C, 1, B), jnp.int32),
            jax.ShapeDtypeStruct((C, D, B), jnp.float32),
            jax.ShapeDtypeStruct((B, C, K), w_q.dtype),
        ],
    )(wq_t, cb_t)
    idx = jnp.transpose(idx_t[:, 0, :], (1, 0))
    w_e = jnp.transpose(we_t, (2, 0, 1))
    w = w_q + jax.lax.stop_gradient(w_e - w_q)
    return (w, w_e, idx, one_hot)


# speedup vs baseline: 1.0773x; 1.0773x over previous
import jax
import jax.numpy as jnp
from jax.experimental import pallas as pl


def _quantize_kernel(wq_ref, cbt_ref, idx_ref, wet_ref):
    NC = wq_ref.shape[0]
    for i in range(NC):
        wq = wq_ref[i]
        cbt = cbt_ref[i]
        B, D = wq.shape
        K = cbt.shape[1]
        wq_sq = jnp.sum(wq * wq, axis=1)
        cb_sq = jnp.sum(cbt * cbt, axis=0)
        cross = jax.lax.dot_general(
            wq, cbt, (((1,), (0,)), ((), ())),
            preferred_element_type=jnp.float32,
            precision=jax.lax.Precision.DEFAULT)
        d = (wq_sq[:, None] + cb_sq[None, :]) - 2.0 * cross
        idx = jnp.argmin(d, axis=1).astype(jnp.int32)
        idx_ref[i, 0] = idx
        kio_t = jax.lax.broadcasted_iota(jnp.int32, (K, 1), 0)
        oh_t = (kio_t == idx[None, :]).astype(jnp.float32)
        wet = jax.lax.dot_general(
            cbt, oh_t, (((1,), (0,)), ((), ())),
            preferred_element_type=jnp.float32,
            precision=jax.lax.Precision.DEFAULT)
        wet_ref[i] = wet


def _onehot_kernel(idx_ref, oh_ref):
    NB = idx_ref.shape[0]
    C = idx_ref.shape[2]
    K = oh_ref.shape[2]
    kio = jax.lax.broadcasted_iota(jnp.int32, (C, K), 1)
    for j in range(NB):
        row = idx_ref[j, 0]
        oh_ref[j] = (kio == row[:, None]).astype(oh_ref.dtype)


def kernel(w_q, codebook):
    B, C, D = w_q.shape
    K = codebook.shape[1]
    wq_t = jnp.transpose(w_q, (1, 0, 2))
    cb_t = jnp.transpose(codebook, (0, 2, 1))
    NC = 4
    idx_t, we_t = pl.pallas_call(
        _quantize_kernel,
        grid=(C // NC,),
        in_specs=[
            pl.BlockSpec((NC, B, D), lambda c: (c, 0, 0)),
            pl.BlockSpec((NC, D, K), lambda c: (c, 0, 0)),
        ],
        out_specs=[
            pl.BlockSpec((NC, 1, B), lambda c: (c, 0, 0)),
            pl.BlockSpec((NC, D, B), lambda c: (c, 0, 0)),
        ],
        out_shape=[
            jax.ShapeDtypeStruct((C, 1, B), jnp.int32),
            jax.ShapeDtypeStruct((C, D, B), jnp.float32),
        ],
    )(wq_t, cb_t)
    idx = jnp.transpose(idx_t[:, 0, :], (1, 0))
    w_e = jnp.transpose(we_t, (2, 0, 1))
    NB = 8
    one_hot = pl.pallas_call(
        _onehot_kernel,
        grid=(B // NB,),
        in_specs=[pl.BlockSpec((NB, 1, C), lambda b: (b, 0, 0))],
        out_specs=pl.BlockSpec((NB, C, K), lambda b: (b, 0, 0)),
        out_shape=jax.ShapeDtypeStruct((B, C, K), w_q.dtype),
    )(idx.reshape(B, 1, C))
    w = w_q + jax.lax.stop_gradient(w_e - w_q)
    return (w, w_e, idx, one_hot)
